# baseline (speedup 1.0000x reference)
import jax
import jax.numpy as jnp
from jax.experimental import pallas as pl
from jax.experimental.pallas import tpu as pltpu

N = 20000
NUM_OBJ_CLS = 151
NUM_REL_CLS = 51
REL_DIM = 4096

BLOCK_N = 800


def _body(obj_ref, vr_ref, wt_ref, b_ref, pred_ref, rel_ref):
    x = obj_ref[...]
    col = jax.lax.broadcasted_iota(jnp.int32, x.shape, 1)
    valid = jnp.logical_and(col >= 1, col < NUM_OBJ_CLS)
    masked = jnp.where(valid, x, -jnp.inf)
    m = jnp.max(masked, axis=1, keepdims=True)
    idx = jnp.min(jnp.where(masked == m, col, NUM_OBJ_CLS), axis=1)
    pred_ref[...] = idx.astype(jnp.int32)[:, None]
    rel = jnp.dot(vr_ref[...], wt_ref[...], preferred_element_type=jnp.float32)
    rel_ref[...] = rel + b_ref[...]


def kernel(obj_logits, vr, W, b):
    wt = W.T
    b2 = b.reshape(1, NUM_REL_CLS)
    grid = (N // BLOCK_N,)
    preds, rel = pl.pallas_call(
        _body,
        grid=grid,
        in_specs=[
            pl.BlockSpec((BLOCK_N, NUM_OBJ_CLS), lambda i: (i, 0)),
            pl.BlockSpec((BLOCK_N, REL_DIM), lambda i: (i, 0)),
            pl.BlockSpec((REL_DIM, NUM_REL_CLS), lambda i: (0, 0)),
            pl.BlockSpec((1, NUM_REL_CLS), lambda i: (0, 0)),
        ],
        out_specs=[
            pl.BlockSpec((BLOCK_N, 1), lambda i: (i, 0)),
            pl.BlockSpec((BLOCK_N, NUM_REL_CLS), lambda i: (i, 0)),
        ],
        out_shape=[
            jax.ShapeDtypeStruct((N, 1), jnp.int32),
            jax.ShapeDtypeStruct((N, NUM_REL_CLS), jnp.float32),
        ],
        compiler_params=pltpu.CompilerParams(
            dimension_semantics=("arbitrary",),
        ),
    )(obj_logits, vr, wt, b2)
    return (obj_logits, preds.reshape(N), rel)
